# baseline (speedup 1.0000x reference)
import functools
from math import log

import jax
import jax.numpy as jnp
from jax import lax
from jax.experimental import pallas as pl
from jax.experimental.pallas import tpu as pltpu
from jax.experimental.pallas import tpu_sc as plsc

N = 10000
E = 320000
C = 128
B = 512

NC = 2
NS = 16
NW = NC * NS
NP = 10240
EP = 327680
ET = EP // NW
K = 128
K2 = 128
CH = 5120
NCH = ET // CH
NBC = CH // K2
RS = NP // NS

@functools.lru_cache(maxsize=None)
def _build_sc_kernels():
    mesh = plsc.VectorSubcoreMesh(
        core_axis_name="c", subcore_axis_name="s",
        num_cores=NC, num_subcores=NS)

    @functools.partial(
        pl.kernel,
        out_type=jax.ShapeDtypeStruct((NC, NP, C), jnp.float32),
        mesh=mesh,
        scratch_types=[
            pltpu.VMEM((ET,), jnp.int32),
            pltpu.VMEM((K, C), jnp.float32),
            pltpu.VMEM((K, C), jnp.float32),
            pltpu.VMEM_SHARED((NP, C), jnp.float32),
        ],
    )
    def sc_pre(dst_hbm, z_hbm, ones_hbm, cnt_hbm,
               dst_v, ones_v, cb_v, cnt_sh):
        c = lax.axis_index("c")
        s = lax.axis_index("s")
        w = c * NS + s

        pltpu.sync_copy(z_hbm.at[pl.ds(0, K)], cb_v)
        for j in range(RS // K):
            pltpu.sync_copy(cb_v, cnt_sh.at[pl.ds(s * RS + j * K, K)])

        pltpu.sync_copy(ones_hbm.at[pl.ds(0, K)], ones_v)
        pltpu.sync_copy(dst_hbm.at[pl.ds(w * ET, ET)], dst_v)
        plsc.subcore_barrier()

        def cnt_body(i, carry):
            pltpu.sync_copy(ones_v, cnt_sh.at[dst_v.at[pl.ds(i * K, K)]],
                            add=True)
            return carry

        lax.fori_loop(0, ET // K, cnt_body, 0)
        plsc.subcore_barrier()

        for j in range(RS // K):
            pltpu.sync_copy(cnt_sh.at[pl.ds(s * RS + j * K, K)], cb_v)
            pltpu.sync_copy(cb_v, cnt_hbm.at[c, pl.ds(s * RS + j * K, K)])

    @functools.partial(
        pl.kernel,
        out_type=jax.ShapeDtypeStruct((NC, NP, C), jnp.float32),
        mesh=mesh,
    scratch_types=[
            pltpu.VMEM((CH,), jnp.int32),
            pltpu.VMEM((CH,), jnp.int32),
            pltpu.VMEM((K2, C), jnp.float32),
            pltpu.VMEM((K2, C), jnp.float32),
            pltpu.SemaphoreType.DMA,
            pltpu.VMEM_SHARED((NP, C), jnp.float32),
        ],
    )
    def sc_scatter(h_hbm, src_hbm, dst_hbm, z_hbm, part_hbm,
                   src_v, dst_v, rows_a, rows_b, sem, agg_sh):
        c = lax.axis_index("c")
        s = lax.axis_index("s")
        w = c * NS + s

        pltpu.sync_copy(z_hbm, rows_a)
        for j in range(RS // K2):
            pltpu.sync_copy(rows_a, agg_sh.at[pl.ds(s * RS + j * K2, K2)])
        plsc.subcore_barrier()

        def chunk(h, carry):
            base = w * ET + h * CH
            pltpu.sync_copy(src_hbm.at[pl.ds(base, CH)], src_v)
            pltpu.sync_copy(dst_hbm.at[pl.ds(base, CH)], dst_v)

            pltpu.async_copy(h_hbm.at[src_v.at[pl.ds(0, K2)]], rows_a, sem)

            def body(j, carry2):
                for b, (buf, other) in enumerate(
                        ((rows_a, rows_b), (rows_b, rows_a))):
                    i = 2 * j + b
                    pltpu.make_async_copy(
                        h_hbm.at[pl.ds(0, K2)], buf, sem).wait()
                    nxt = i + 1

                    @pl.when(nxt < NBC)
                    def _():
                        pltpu.async_copy(
                            h_hbm.at[src_v.at[pl.ds(nxt * K2, K2)]],
                            other, sem)

                    pltpu.sync_copy(buf,
                                    agg_sh.at[dst_v.at[pl.ds(i * K2, K2)]],
                                    add=True)
                return carry2

            lax.fori_loop(0, NBC // 2, body, 0)
            return carry

        lax.fori_loop(0, NCH, chunk, 0)
        plsc.subcore_barrier()

        for j in range(RS // K2):
            pltpu.sync_copy(agg_sh.at[pl.ds(s * RS + j * K2, K2)], rows_a)
            pltpu.sync_copy(rows_a, part_hbm.at[c, pl.ds(s * RS + j * K2, K2)])

    return sc_pre, sc_scatter


_RB = 1024


def _enc_body(x_ref, bat_ref, nt_ref, seed_ref, we_ref, be_ref, wt_ref,
              o_ref):
    oh = (bat_ref[...] ==
          lax.broadcasted_iota(jnp.int32, (_RB, B), 1)).astype(jnp.float32)
    stg = jnp.dot(oh, seed_ref[...], preferred_element_type=jnp.float32)
    rel = (stg[:, 0:1] - nt_ref[...].astype(jnp.float32)) / 86400.0
    half = C // 2
    kidx = lax.broadcasted_iota(jnp.int32, (1, half), 1).astype(jnp.float32)
    freqs = jnp.exp(kidx * (-log(10000.0) / half))
    ang = rel * freqs
    pe = jnp.concatenate([jnp.sin(ang), jnp.cos(ang)], axis=1)
    o_ref[...] = (
        jnp.dot(x_ref[...], we_ref[...], preferred_element_type=jnp.float32)
        + be_ref[...]
        + jnp.dot(pe, wt_ref[...], preferred_element_type=jnp.float32))


def _encode(x_p, bat2d, nt2d, seed2d, W_enc, b_enc2d, W_time):
    grid = (NP // _RB,)
    return pl.pallas_call(
        _enc_body,
        grid=grid,
        in_specs=[
            pl.BlockSpec((_RB, C), lambda i: (i, 0)),
            pl.BlockSpec((_RB, 1), lambda i: (i, 0)),
            pl.BlockSpec((_RB, 1), lambda i: (i, 0)),
            pl.BlockSpec((B, C), lambda i: (0, 0)),
            pl.BlockSpec((C, C), lambda i: (0, 0)),
            pl.BlockSpec((1, C), lambda i: (0, 0)),
            pl.BlockSpec((C, C), lambda i: (0, 0)),
        ],
        out_specs=pl.BlockSpec((_RB, C), lambda i: (i, 0)),
        out_shape=jax.ShapeDtypeStruct((NP, C), jnp.float32),
    )(x_p, bat2d, nt2d, seed2d, W_enc, b_enc2d, W_time)


def _layer_body(h_ref, p_ref, cnt_ref, ws_ref, wn_ref, b_ref, o_ref):
    pv = p_ref[0] + p_ref[1]
    cntf = cnt_ref[0, :, 0:1] + cnt_ref[1, :, 0:1]
    mean = pv / jnp.maximum(cntf, 1.0)
    o_ref[...] = jax.nn.relu(
        jnp.dot(h_ref[...], ws_ref[...], preferred_element_type=jnp.float32)
        + jnp.dot(mean, wn_ref[...], preferred_element_type=jnp.float32)
        + b_ref[...])


def _sage_layer(h, part, cnt, Ws, Wn, b2d):
    grid = (NP // _RB,)
    return pl.pallas_call(
        _layer_body,
        grid=grid,
        in_specs=[
            pl.BlockSpec((_RB, C), lambda i: (i, 0)),
            pl.BlockSpec((NC, _RB, C), lambda i: (0, i, 0)),
            pl.BlockSpec((NC, _RB, C), lambda i: (0, i, 0)),
            pl.BlockSpec((C, C), lambda i: (0, 0)),
            pl.BlockSpec((C, C), lambda i: (0, 0)),
            pl.BlockSpec((1, C), lambda i: (0, 0)),
        ],
        out_specs=pl.BlockSpec((_RB, C), lambda i: (i, 0)),
        out_shape=jax.ShapeDtypeStruct((NP, C), jnp.float32),
    )(h, part, cnt, Ws, Wn, b2d)


def _final_body(h_ref, p_ref, cnt_ref, ws_ref, wn_ref, b_ref,
                wh_ref, bh_ref, o_ref):
    pv = p_ref[0] + p_ref[1]
    cntf = cnt_ref[0, :, 0:1] + cnt_ref[1, :, 0:1]
    mean = pv / jnp.maximum(cntf, 1.0)
    h2 = jax.nn.relu(
        jnp.dot(h_ref[...], ws_ref[...], preferred_element_type=jnp.float32)
        + jnp.dot(mean, wn_ref[...], preferred_element_type=jnp.float32)
        + b_ref[...])
    o_ref[...] = (
        jnp.dot(h2, wh_ref[...], preferred_element_type=jnp.float32)
        + bh_ref[...])


def _final_layer(h, part, cnt, Ws, Wn, b2d, Wh_pad, bh_pad):
    return pl.pallas_call(
        _final_body,
        grid=(1,),
        in_specs=[
            pl.BlockSpec((B, C), lambda i: (0, 0)),
            pl.BlockSpec((NC, B, C), lambda i: (0, 0, 0)),
            pl.BlockSpec((NC, B, C), lambda i: (0, 0, 0)),
            pl.BlockSpec((C, C), lambda i: (0, 0)),
            pl.BlockSpec((C, C), lambda i: (0, 0)),
            pl.BlockSpec((1, C), lambda i: (0, 0)),
            pl.BlockSpec((C, C), lambda i: (0, 0)),
            pl.BlockSpec((1, C), lambda i: (0, 0)),
        ],
        out_specs=pl.BlockSpec((B, C), lambda i: (0, 0)),
        out_shape=jax.ShapeDtypeStruct((B, C), jnp.float32),
    )(h[:B], part[:, :B], cnt[:, :B], Ws, Wn, b2d, Wh_pad, bh_pad)


def kernel(x, edge_index, seed_time, node_time, batch,
           W_enc, b_enc, W_time,
           W1_self, W1_neigh, b1,
           W2_self, W2_neigh, b2,
           W_head, b_head):
    f32 = jnp.float32
    i32 = jnp.int32

    src = edge_index[0].astype(i32)
    dst = edge_index[1].astype(i32)
    src_p = jnp.concatenate([src, jnp.zeros((EP - E,), i32)])
    pad_dst = N + (jnp.arange(EP - E, dtype=i32) % (NP - N))
    dst_p = jnp.concatenate([dst, pad_dst])

    seed2d = jnp.broadcast_to(
        seed_time.astype(f32).reshape(B, 1), (B, C))
    bat2d = jnp.concatenate(
        [batch.astype(i32), jnp.zeros((NP - N,), i32)]).reshape(NP, 1)
    nt2d = jnp.concatenate(
        [node_time.astype(i32), jnp.zeros((NP - N,), i32)]).reshape(NP, 1)
    x_p = jnp.concatenate([x, jnp.zeros((NP - N, C), f32)])

    zrow = jnp.zeros((K2, C), f32)
    onesrow = jnp.ones((K2, C), f32)

    b_enc2d = b_enc.reshape(1, C)
    b12d = b1.reshape(1, C)
    b22d = b2.reshape(1, C)
    Wh_pad = jnp.concatenate([W_head, jnp.zeros((C, C - 1), f32)], axis=1)
    bh_pad = jnp.concatenate([b_head, jnp.zeros((C - 1,), f32)]).reshape(1, C)

    sc_pre, sc_scatter = _build_sc_kernels()
    cnt = sc_pre(dst_p, zrow, onesrow)
    h0 = _encode(x_p, bat2d, nt2d, seed2d, W_enc, b_enc2d, W_time)
    part1 = sc_scatter(h0, src_p, dst_p, zrow)
    h1 = _sage_layer(h0, part1, cnt, W1_self, W1_neigh, b12d)
    part2 = sc_scatter(h1, src_p, dst_p, zrow)
    out_full = _final_layer(h1, part2, cnt, W2_self, W2_neigh, b22d,
                            Wh_pad, bh_pad)
    return out_full[:, :1]


# speedup vs baseline: 1.0394x; 1.0394x over previous
import functools
from math import log

import jax
import jax.numpy as jnp
from jax import lax
from jax.experimental import pallas as pl
from jax.experimental.pallas import tpu as pltpu
from jax.experimental.pallas import tpu_sc as plsc

N = 10000
E = 320000
C = 128
B = 512

NC = 2
NS = 16
NW = NC * NS
NP = 10240
EP = 327680
ET = EP // NW
K = 128
K2 = 128
KG = 64
NBUF = 4
CH = 5120
NCH = ET // CH
NBC = CH // K2
RS = NP // NS

@functools.lru_cache(maxsize=None)
def _build_sc_kernels():
    mesh = plsc.VectorSubcoreMesh(
        core_axis_name="c", subcore_axis_name="s",
        num_cores=NC, num_subcores=NS)

    @functools.partial(
        pl.kernel,
        out_type=jax.ShapeDtypeStruct((NC, NP, C), jnp.float32),
        mesh=mesh,
        scratch_types=[
            pltpu.VMEM((ET,), jnp.int32),
            pltpu.VMEM((K, C), jnp.float32),
            pltpu.VMEM((K, C), jnp.float32),
            pltpu.VMEM_SHARED((NP, C), jnp.float32),
        ],
    )
    def sc_pre(dst_hbm, z_hbm, ones_hbm, cnt_hbm,
               dst_v, ones_v, cb_v, cnt_sh):
        c = lax.axis_index("c")
        s = lax.axis_index("s")
        w = c * NS + s

        pltpu.sync_copy(z_hbm.at[pl.ds(0, K)], cb_v)
        for j in range(RS // K):
            pltpu.sync_copy(cb_v, cnt_sh.at[pl.ds(s * RS + j * K, K)])

        pltpu.sync_copy(ones_hbm.at[pl.ds(0, K)], ones_v)
        pltpu.sync_copy(dst_hbm.at[pl.ds(w * ET, ET)], dst_v)
        plsc.subcore_barrier()

        def cnt_body(i, carry):
            pltpu.sync_copy(ones_v, cnt_sh.at[dst_v.at[pl.ds(i * K, K)]],
                            add=True)
            return carry

        lax.fori_loop(0, ET // K, cnt_body, 0)
        plsc.subcore_barrier()

        for j in range(RS // K):
            pltpu.sync_copy(cnt_sh.at[pl.ds(s * RS + j * K, K)], cb_v)
            pltpu.sync_copy(cb_v, cnt_hbm.at[c, pl.ds(s * RS + j * K, K)])

    @functools.partial(
        pl.kernel,
        out_type=jax.ShapeDtypeStruct((NC, NP, C), jnp.float32),
        mesh=mesh,
    scratch_types=[
            pltpu.VMEM((CH,), jnp.int32),
            pltpu.VMEM((CH,), jnp.int32),
            pltpu.VMEM((KG, C), jnp.float32),
            pltpu.VMEM((KG, C), jnp.float32),
            pltpu.VMEM((KG, C), jnp.float32),
            pltpu.VMEM((KG, C), jnp.float32),
            pltpu.SemaphoreType.DMA,
            pltpu.VMEM_SHARED((NP, C), jnp.float32),
        ],
    )
    def sc_scatter(h_hbm, src_hbm, dst_hbm, z_hbm, part_hbm,
                   src_v, dst_v, r0, r1, r2, r3, sem, agg_sh):
        c = lax.axis_index("c")
        s = lax.axis_index("s")
        w = c * NS + s
        bufs = (r0, r1, r2, r3)
        NG = CH // KG

        pltpu.sync_copy(z_hbm.at[pl.ds(0, KG)], r0)
        for j in range(RS // KG):
            pltpu.sync_copy(r0, agg_sh.at[pl.ds(s * RS + j * KG, KG)])
        plsc.subcore_barrier()

        def chunk(h, carry):
            base = w * ET + h * CH
            pltpu.sync_copy(src_hbm.at[pl.ds(base, CH)], src_v)
            pltpu.sync_copy(dst_hbm.at[pl.ds(base, CH)], dst_v)

            for b in range(NBUF - 1):
                pltpu.async_copy(
                    h_hbm.at[src_v.at[pl.ds(b * KG, KG)]], bufs[b], sem)

            def body(j, carry2):
                for b in range(NBUF):
                    i = NBUF * j + b
                    nxt = i + NBUF - 1

                    @pl.when(nxt < NG)
                    def _():
                        pltpu.async_copy(
                            h_hbm.at[src_v.at[pl.ds(nxt * KG, KG)]],
                            bufs[(b + NBUF - 1) % NBUF], sem)

                    pltpu.make_async_copy(
                        h_hbm.at[pl.ds(0, KG)], bufs[b], sem).wait()
                    pltpu.sync_copy(bufs[b],
                                    agg_sh.at[dst_v.at[pl.ds(i * KG, KG)]],
                                    add=True)
                return carry2

            lax.fori_loop(0, NG // NBUF, body, 0)
            return carry

        lax.fori_loop(0, NCH, chunk, 0)
        plsc.subcore_barrier()

        for j in range(RS // KG):
            pltpu.sync_copy(agg_sh.at[pl.ds(s * RS + j * KG, KG)], r0)
            pltpu.sync_copy(r0, part_hbm.at[c, pl.ds(s * RS + j * KG, KG)])

    return sc_pre, sc_scatter


_RB = 1024


def _enc_body(x_ref, bat_ref, nt_ref, seed_ref, we_ref, be_ref, wt_ref,
              o_ref):
    oh = (bat_ref[...] ==
          lax.broadcasted_iota(jnp.int32, (_RB, B), 1)).astype(jnp.float32)
    stg = jnp.dot(oh, seed_ref[...], preferred_element_type=jnp.float32)
    rel = (stg[:, 0:1] - nt_ref[...].astype(jnp.float32)) / 86400.0
    half = C // 2
    kidx = lax.broadcasted_iota(jnp.int32, (1, half), 1).astype(jnp.float32)
    freqs = jnp.exp(kidx * (-log(10000.0) / half))
    ang = rel * freqs
    pe = jnp.concatenate([jnp.sin(ang), jnp.cos(ang)], axis=1)
    o_ref[...] = (
        jnp.dot(x_ref[...], we_ref[...], preferred_element_type=jnp.float32)
        + be_ref[...]
        + jnp.dot(pe, wt_ref[...], preferred_element_type=jnp.float32))


def _encode(x_p, bat2d, nt2d, seed2d, W_enc, b_enc2d, W_time):
    grid = (NP // _RB,)
    return pl.pallas_call(
        _enc_body,
        grid=grid,
        in_specs=[
            pl.BlockSpec((_RB, C), lambda i: (i, 0)),
            pl.BlockSpec((_RB, 1), lambda i: (i, 0)),
            pl.BlockSpec((_RB, 1), lambda i: (i, 0)),
            pl.BlockSpec((B, C), lambda i: (0, 0)),
            pl.BlockSpec((C, C), lambda i: (0, 0)),
            pl.BlockSpec((1, C), lambda i: (0, 0)),
            pl.BlockSpec((C, C), lambda i: (0, 0)),
        ],
        out_specs=pl.BlockSpec((_RB, C), lambda i: (i, 0)),
        out_shape=jax.ShapeDtypeStruct((NP, C), jnp.float32),
    )(x_p, bat2d, nt2d, seed2d, W_enc, b_enc2d, W_time)


def _layer_body(h_ref, p_ref, cnt_ref, ws_ref, wn_ref, b_ref, o_ref):
    pv = p_ref[0] + p_ref[1]
    cntf = cnt_ref[0, :, 0:1] + cnt_ref[1, :, 0:1]
    mean = pv / jnp.maximum(cntf, 1.0)
    o_ref[...] = jax.nn.relu(
        jnp.dot(h_ref[...], ws_ref[...], preferred_element_type=jnp.float32)
        + jnp.dot(mean, wn_ref[...], preferred_element_type=jnp.float32)
        + b_ref[...])


def _sage_layer(h, part, cnt, Ws, Wn, b2d):
    grid = (NP // _RB,)
    return pl.pallas_call(
        _layer_body,
        grid=grid,
        in_specs=[
            pl.BlockSpec((_RB, C), lambda i: (i, 0)),
            pl.BlockSpec((NC, _RB, C), lambda i: (0, i, 0)),
            pl.BlockSpec((NC, _RB, C), lambda i: (0, i, 0)),
            pl.BlockSpec((C, C), lambda i: (0, 0)),
            pl.BlockSpec((C, C), lambda i: (0, 0)),
            pl.BlockSpec((1, C), lambda i: (0, 0)),
        ],
        out_specs=pl.BlockSpec((_RB, C), lambda i: (i, 0)),
        out_shape=jax.ShapeDtypeStruct((NP, C), jnp.float32),
    )(h, part, cnt, Ws, Wn, b2d)


def _final_body(h_ref, p_ref, cnt_ref, ws_ref, wn_ref, b_ref,
                wh_ref, bh_ref, o_ref):
    pv = p_ref[0] + p_ref[1]
    cntf = cnt_ref[0, :, 0:1] + cnt_ref[1, :, 0:1]
    mean = pv / jnp.maximum(cntf, 1.0)
    h2 = jax.nn.relu(
        jnp.dot(h_ref[...], ws_ref[...], preferred_element_type=jnp.float32)
        + jnp.dot(mean, wn_ref[...], preferred_element_type=jnp.float32)
        + b_ref[...])
    o_ref[...] = (
        jnp.dot(h2, wh_ref[...], preferred_element_type=jnp.float32)
        + bh_ref[...])


def _final_layer(h, part, cnt, Ws, Wn, b2d, Wh_pad, bh_pad):
    return pl.pallas_call(
        _final_body,
        grid=(1,),
        in_specs=[
            pl.BlockSpec((B, C), lambda i: (0, 0)),
            pl.BlockSpec((NC, B, C), lambda i: (0, 0, 0)),
            pl.BlockSpec((NC, B, C), lambda i: (0, 0, 0)),
            pl.BlockSpec((C, C), lambda i: (0, 0)),
            pl.BlockSpec((C, C), lambda i: (0, 0)),
            pl.BlockSpec((1, C), lambda i: (0, 0)),
            pl.BlockSpec((C, C), lambda i: (0, 0)),
            pl.BlockSpec((1, C), lambda i: (0, 0)),
        ],
        out_specs=pl.BlockSpec((B, C), lambda i: (0, 0)),
        out_shape=jax.ShapeDtypeStruct((B, C), jnp.float32),
    )(h[:B], part[:, :B], cnt[:, :B], Ws, Wn, b2d, Wh_pad, bh_pad)


def kernel(x, edge_index, seed_time, node_time, batch,
           W_enc, b_enc, W_time,
           W1_self, W1_neigh, b1,
           W2_self, W2_neigh, b2,
           W_head, b_head):
    f32 = jnp.float32
    i32 = jnp.int32

    src = edge_index[0].astype(i32)
    dst = edge_index[1].astype(i32)
    src_p = jnp.concatenate([src, jnp.zeros((EP - E,), i32)])
    pad_dst = N + (jnp.arange(EP - E, dtype=i32) % (NP - N))
    dst_p = jnp.concatenate([dst, pad_dst])

    seed2d = jnp.broadcast_to(
        seed_time.astype(f32).reshape(B, 1), (B, C))
    bat2d = jnp.concatenate(
        [batch.astype(i32), jnp.zeros((NP - N,), i32)]).reshape(NP, 1)
    nt2d = jnp.concatenate(
        [node_time.astype(i32), jnp.zeros((NP - N,), i32)]).reshape(NP, 1)
    x_p = jnp.concatenate([x, jnp.zeros((NP - N, C), f32)])

    zrow = jnp.zeros((K2, C), f32)
    onesrow = jnp.ones((K2, C), f32)

    b_enc2d = b_enc.reshape(1, C)
    b12d = b1.reshape(1, C)
    b22d = b2.reshape(1, C)
    Wh_pad = jnp.concatenate([W_head, jnp.zeros((C, C - 1), f32)], axis=1)
    bh_pad = jnp.concatenate([b_head, jnp.zeros((C - 1,), f32)]).reshape(1, C)

    sc_pre, sc_scatter = _build_sc_kernels()
    cnt = sc_pre(dst_p, zrow, onesrow)
    h0 = _encode(x_p, bat2d, nt2d, seed2d, W_enc, b_enc2d, W_time)
    part1 = sc_scatter(h0, src_p, dst_p, zrow)
    h1 = _sage_layer(h0, part1, cnt, W1_self, W1_neigh, b12d)
    part2 = sc_scatter(h1, src_p, dst_p, zrow)
    out_full = _final_layer(h1, part2, cnt, W2_self, W2_neigh, b22d,
                            Wh_pad, bh_pad)
    return out_full[:, :1]
